# baseline (speedup 1.0000x reference)
import functools

import jax
import jax.numpy as jnp
from jax import lax
from jax.experimental import pallas as pl
from jax.experimental.pallas import tpu as pltpu
from jax.experimental.pallas import tpu_sc as plsc

EMBED_DIM = 64
_info = plsc.get_sparse_core_info()
_NC, _NS = _info.num_cores, _info.num_subcores
_NW = _NC * _NS

_CHUNK = 256
_NBUF = 5
_DMA_F = 144
_DMA_LAST = 112


def _make_gather(B: int, V: int):
  b_per_w = B // _NW
  n_groups = 18
  s_rows = _NBUF * _CHUNK * n_groups
  assert s_rows + _DMA_F * (n_groups - 1) + _DMA_LAST == b_per_w
  mesh = plsc.VectorSubcoreMesh(core_axis_name="c", subcore_axis_name="s")

  @functools.partial(
      pl.kernel,
      mesh=mesh,
      out_type=jax.ShapeDtypeStruct((B, EMBED_DIM), jnp.float32),
      scratch_types=[
          pltpu.VMEM((b_per_w,), jnp.int32),
          [pltpu.VMEM((_CHUNK, EMBED_DIM), jnp.float32) for _ in range(_NBUF)],
          [pltpu.SemaphoreType.DMA for _ in range(_NBUF)],
          [pltpu.SemaphoreType.DMA for _ in range(_NBUF)],
          pltpu.SemaphoreType.DMA,
      ],
      compiler_params=pltpu.CompilerParams(use_tc_tiling_on_sc=False),
  )
  def gather_kernel(idx_hbm, table_hbm, out_hbm, idx_v, rows, sg, sw, sd):
    wid = lax.axis_index("s") * _NC + lax.axis_index("c")
    base = wid * b_per_w
    dma_base = s_rows

    pltpu.sync_copy(idx_hbm.at[pl.ds(base, b_per_w)], idx_v)

    def gather(c, b):
      return pltpu.make_async_copy(
          table_hbm.at[idx_v.at[pl.ds(c * _CHUNK, _CHUNK)]], rows[b], sg[b])

    def write(c, b):
      return pltpu.make_async_copy(
          rows[b], out_hbm.at[pl.ds(base + c * _CHUNK, _CHUNK)], sw[b])

    def row_copy(s, r):
      return pltpu.make_async_copy(
          table_hbm.at[pl.ds(s, 1)], out_hbm.at[pl.ds(r, 1)], sd)

    def dma_fire(off, count):
      for t in range(count // 16):
        v = idx_v[pl.ds(off + t * 16, 16)]
        for k in range(16):
          row_copy(v[k], base + off + t * 16 + k).start()

    def dma_drain(count):
      for _ in range(count):
        row_copy(0, base).wait()

    for b in range(_NBUF):
      gather(b, b).start()

    def group(j, dma_count, refill):
      dma_fire(dma_base + j * _DMA_F, dma_count)
      for b in range(_NBUF):
        c = j * _NBUF + b
        gather(c, b).wait()
        write(c, b).start()
        write(c, b).wait()
        if refill:
          gather(c + _NBUF, b).start()
      dma_drain(dma_count)

    lax.fori_loop(0, n_groups - 1,
                  lambda j, c: (group(j, _DMA_F, True), c)[1], 0)
    group(n_groups - 1, _DMA_LAST, False)

  return gather_kernel


def kernel(input, emb):
  B0, B1 = input.shape
  V, D = emb.shape
  assert D == EMBED_DIM
  flat_idx = input.reshape(B0 * B1).astype(jnp.int32)
  out = _make_gather(B0 * B1, V)(flat_idx, emb)
  return out.reshape(B0, B1, D)


# speedup vs baseline: 1.3968x; 1.3968x over previous
import functools

import jax
import jax.numpy as jnp
from jax import lax
from jax.experimental import pallas as pl
from jax.experimental.pallas import tpu as pltpu
from jax.experimental.pallas import tpu_sc as plsc

EMBED_DIM = 64
_info = plsc.get_sparse_core_info()
_NC, _NS = _info.num_cores, _info.num_subcores
_NW = _NC * _NS

_CHUNK = 256
_NBUF = 5


def _make_gather(B: int, V: int):
  b_per_w = B // _NW
  n_chunks = b_per_w // _CHUNK
  n_groups = n_chunks // _NBUF
  mesh = plsc.VectorSubcoreMesh(core_axis_name="c", subcore_axis_name="s")

  @functools.partial(
      pl.kernel,
      mesh=mesh,
      out_type=jax.ShapeDtypeStruct((B, EMBED_DIM), jnp.float32),
      scratch_types=[
          pltpu.VMEM((b_per_w,), jnp.int32),
          [pltpu.VMEM((_CHUNK, EMBED_DIM), jnp.float32) for _ in range(_NBUF)],
          [pltpu.SemaphoreType.DMA for _ in range(_NBUF)],
          [pltpu.SemaphoreType.DMA for _ in range(_NBUF)],
      ],
      compiler_params=pltpu.CompilerParams(use_tc_tiling_on_sc=False),
  )
  def gather_kernel(idx_hbm, table_hbm, out_hbm, idx_v, rows, sg, sw):
    wid = lax.axis_index("s") * _NC + lax.axis_index("c")
    base = wid * b_per_w

    pltpu.sync_copy(idx_hbm.at[pl.ds(base, b_per_w)], idx_v)

    def gather(c, b):
      return pltpu.make_async_copy(
          table_hbm.at[idx_v.at[pl.ds(c * _CHUNK, _CHUNK)]], rows[b], sg[b])

    def write(c, b):
      return pltpu.make_async_copy(
          rows[b], out_hbm.at[pl.ds(base + c * _CHUNK, _CHUNK)], sw[b])

    for b in range(_NBUF):
      gather(b, b).start()

    def group(j, refill):
      for b in range(_NBUF):
        c = j * _NBUF + b
        gather(c, b).wait()
        write(c, b).start()
        write(c, b).wait()
        if refill:
          gather(c + _NBUF, b).start()

    lax.fori_loop(0, n_groups - 1, lambda j, c: (group(j, True), c)[1], 0)
    group(n_groups - 1, False)

  return gather_kernel


def kernel(input, emb):
  B0, B1 = input.shape
  V, D = emb.shape
  assert D == EMBED_DIM
  flat_idx = input.reshape(B0 * B1).astype(jnp.int32)
  out = _make_gather(B0 * B1, V)(flat_idx, emb)
  return out.reshape(B0, B1, D)
